# Initial kernel scaffold; baseline (speedup 1.0000x reference)
import functools

import jax
import jax.numpy as jnp
from jax import lax
from jax.experimental import pallas as pl
from jax.experimental.pallas import tpu as pltpu
from jax.experimental.pallas import tpu_sc as plsc

NC = 2
NS = 16
NW = NC * NS

D = 32
B_TOTAL = 16384 * 50
B_PER_W = B_TOTAL // NW
CHUNK = 2560
N_CHUNKS = B_PER_W // CHUNK


def _make_kernel():
  mesh = plsc.VectorSubcoreMesh(core_axis_name="c", subcore_axis_name="s")

  @functools.partial(
      pl.kernel,
      mesh=mesh,
      out_type=jax.ShapeDtypeStruct((B_TOTAL, D), jnp.float32),
      scratch_types=[
          pltpu.VMEM((CHUNK,), jnp.int32),
          pltpu.VMEM((CHUNK, D), jnp.float32),
          pltpu.SemaphoreType.DMA,
      ],
  )
  def gather_kernel(idx_hbm, table_hbm, out_hbm, idx_v, rows_v, sem):
    wid = lax.axis_index("s") * NC + lax.axis_index("c")
    base_w = wid * B_PER_W

    def body(i, carry):
      base = base_w + i * CHUNK
      pltpu.sync_copy(idx_hbm.at[pl.ds(base, CHUNK)], idx_v)
      pltpu.async_copy(table_hbm.at[idx_v], rows_v, sem).wait()
      pltpu.sync_copy(rows_v, out_hbm.at[pl.ds(base, CHUNK)])
      return carry

    lax.fori_loop(0, N_CHUNKS, body, 0)

  return gather_kernel


_gather = _make_kernel()


@jax.jit
def kernel(weights, indices):
  idx_flat = indices.reshape(-1).astype(jnp.int32)
  out = _gather(idx_flat, weights)
  return out.reshape(indices.shape + (D,))


# speedup vs baseline: 1.1082x; 1.1082x over previous
import functools

import jax
import jax.numpy as jnp
from jax import lax
from jax.experimental import pallas as pl
from jax.experimental.pallas import tpu as pltpu
from jax.experimental.pallas import tpu_sc as plsc

NC = 2
NS = 16
NW = NC * NS

D = 32
B_TOTAL = 16384 * 50
B_PER_W = B_TOTAL // NW
CHUNK = 2560
N_CHUNKS = B_PER_W // CHUNK


def _make_kernel():
  mesh = plsc.VectorSubcoreMesh(core_axis_name="c", subcore_axis_name="s")

  @functools.partial(
      pl.kernel,
      mesh=mesh,
      out_type=jax.ShapeDtypeStruct((B_TOTAL, D), jnp.float32),
      scratch_types=[
          pltpu.VMEM((CHUNK,), jnp.int32),
          pltpu.VMEM((CHUNK, D), jnp.float32),
          pltpu.SemaphoreType.DMA,
      ],
      compiler_params=pltpu.CompilerParams(use_tc_tiling_on_sc=False),
  )
  def gather_kernel(idx_hbm, table_hbm, out_hbm, idx_v, rows_v, sem):
    wid = lax.axis_index("s") * NC + lax.axis_index("c")
    base_w = wid * B_PER_W

    def body(i, carry):
      base = base_w + i * CHUNK
      pltpu.sync_copy(idx_hbm.at[pl.ds(base, CHUNK)], idx_v)
      pltpu.async_copy(table_hbm.at[idx_v], rows_v, sem).wait()
      pltpu.sync_copy(rows_v, out_hbm.at[pl.ds(base, CHUNK)])
      return carry

    lax.fori_loop(0, N_CHUNKS, body, 0)

  return gather_kernel


_gather = _make_kernel()


@jax.jit
def kernel(weights, indices):
  idx_flat = indices.reshape(-1).astype(jnp.int32)
  out = _gather(idx_flat, weights)
  return out.reshape(indices.shape + (D,))


# speedup vs baseline: 1.1083x; 1.0001x over previous
import functools

import jax
import jax.numpy as jnp
from jax import lax
from jax.experimental import pallas as pl
from jax.experimental.pallas import tpu as pltpu
from jax.experimental.pallas import tpu_sc as plsc

NC = 2
NS = 16
NW = NC * NS

D = 32
B_TOTAL = 16384 * 50
B_PER_W = B_TOTAL // NW
NBUF = 4
CHUNK = 640
N_CHUNKS = B_PER_W // CHUNK
N_GROUPS = N_CHUNKS // NBUF


def _make_kernel():
  mesh = plsc.VectorSubcoreMesh(core_axis_name="c", subcore_axis_name="s")

  @functools.partial(
      pl.kernel,
      mesh=mesh,
      out_type=jax.ShapeDtypeStruct((B_TOTAL, D), jnp.float32),
      scratch_types=[
          pltpu.VMEM((B_PER_W,), jnp.int32),
          *[pltpu.VMEM((CHUNK, D), jnp.float32) for _ in range(NBUF)],
          *[pltpu.SemaphoreType.DMA for _ in range(2 * NBUF)],
      ],
      compiler_params=pltpu.CompilerParams(use_tc_tiling_on_sc=False),
  )
  def gather_kernel(idx_hbm, table_hbm, out_hbm, idx_all, *bufs_and_sems):
    rows = bufs_and_sems[:NBUF]
    sem_g = bufs_and_sems[NBUF:2 * NBUF]
    sem_o = bufs_and_sems[2 * NBUF:]

    wid = lax.axis_index("s") * NC + lax.axis_index("c")
    base_w = wid * B_PER_W

    pltpu.sync_copy(idx_hbm.at[pl.ds(base_w, B_PER_W)], idx_all)

    def gather(g, b):
      src = table_hbm.at[idx_all.at[pl.ds(pl.multiple_of(g * CHUNK, 8), CHUNK)]]
      return pltpu.make_async_copy(src, rows[b], sem_g[b])

    def store(g, b):
      dst = out_hbm.at[pl.ds(base_w + g * CHUNK, CHUNK)]
      return pltpu.make_async_copy(rows[b], dst, sem_o[b])

    for b in range(NBUF):
      gather(b, b).start()

    def body(j, carry):
      for b in range(NBUF):
        g = j * NBUF + b
        gather(g, b).wait()
        store(g, b).start()
      for b in range(NBUF):
        g = j * NBUF + b
        store(g, b).wait()
        gather(g + NBUF, b).start()
      return carry

    lax.fori_loop(0, N_GROUPS - 1, body, 0)

    jl = N_GROUPS - 1
    for b in range(NBUF):
      g = jl * NBUF + b
      gather(g, b).wait()
      store(g, b).start()
    for b in range(NBUF):
      g = jl * NBUF + b
      store(g, b).wait()

  return gather_kernel


_gather = _make_kernel()


@jax.jit
def kernel(weights, indices):
  idx_flat = indices.reshape(-1).astype(jnp.int32)
  out = _gather(idx_flat, weights)
  return out.reshape(indices.shape + (D,))


# speedup vs baseline: 1.7975x; 1.6219x over previous
import functools

import jax
import jax.numpy as jnp
from jax import lax
from jax.experimental import pallas as pl
from jax.experimental.pallas import tpu as pltpu
from jax.experimental.pallas import tpu_sc as plsc

NC = 2
NS = 16
NW = NC * NS

D = 32
NB = 16384
SEQ = 50
B_TOTAL = NB * SEQ
B_PER_W = B_TOTAL // NW
NB_PER_W = NB // NW
BATCH_CHUNK = 16
CHUNK = BATCH_CHUNK * SEQ
NBUF = 2
N_CHUNKS = NB_PER_W // BATCH_CHUNK
N_GROUPS = N_CHUNKS // NBUF


def _make_kernel():
  mesh = plsc.VectorSubcoreMesh(core_axis_name="c", subcore_axis_name="s")

  @functools.partial(
      pl.kernel,
      mesh=mesh,
      out_type=jax.ShapeDtypeStruct((NB, SEQ, D), jnp.float32),
      scratch_types=[
          pltpu.VMEM((B_PER_W,), jnp.int32),
          *[pltpu.VMEM((CHUNK, D), jnp.float32) for _ in range(NBUF)],
          *[pltpu.SemaphoreType.DMA for _ in range(2 * NBUF)],
      ],
      compiler_params=pltpu.CompilerParams(use_tc_tiling_on_sc=False),
  )
  def gather_kernel(idx_hbm, table_hbm, out_hbm, idx_all, *bufs_and_sems):
    rows = bufs_and_sems[:NBUF]
    sem_g = bufs_and_sems[NBUF:2 * NBUF]
    sem_o = bufs_and_sems[2 * NBUF:]

    wid = lax.axis_index("s") * NC + lax.axis_index("c")
    base_w = wid * B_PER_W
    base_b = wid * NB_PER_W

    pltpu.sync_copy(idx_hbm.at[pl.ds(base_w, B_PER_W)], idx_all)

    def gather(g, b):
      src = table_hbm.at[idx_all.at[pl.ds(pl.multiple_of(g * CHUNK, 8), CHUNK)]]
      return pltpu.make_async_copy(src, rows[b], sem_g[b])

    def store_all(g, b):
      bb = base_b + g * BATCH_CHUNK
      for j in range(BATCH_CHUNK):
        pltpu.make_async_copy(
            rows[b].at[pl.ds(j * SEQ, SEQ)], out_hbm.at[bb + j], sem_o[b]
        ).start()

    def wait_all(g, b):
      bb = base_b + g * BATCH_CHUNK
      for j in range(BATCH_CHUNK):
        pltpu.make_async_copy(
            rows[b].at[pl.ds(j * SEQ, SEQ)], out_hbm.at[bb + j], sem_o[b]
        ).wait()

    for b in range(NBUF):
      gather(b, b).start()

    def body(j, carry):
      for b in range(NBUF):
        g = j * NBUF + b
        gather(g, b).wait()
        store_all(g, b)
      for b in range(NBUF):
        g = j * NBUF + b
        wait_all(g, b)
        gather(g + NBUF, b).start()
      return carry

    lax.fori_loop(0, N_GROUPS - 1, body, 0)

    jl = N_GROUPS - 1
    for b in range(NBUF):
      g = jl * NBUF + b
      gather(g, b).wait()
      store_all(g, b)
    for b in range(NBUF):
      wait_all(jl * NBUF + b, b)

  return gather_kernel


_gather = _make_kernel()


@jax.jit
def kernel(weights, indices):
  idx_flat = indices.reshape(-1).astype(jnp.int32)
  return _gather(idx_flat, weights)
